# Initial kernel scaffold; baseline (speedup 1.0000x reference)
import functools

import jax
import jax.numpy as jnp
from jax import lax
from jax.experimental import pallas as pl
from jax.experimental.pallas import tpu as pltpu
from jax.experimental.pallas import tpu_sc as plsc

D = 32
NC = 2
NS = 16
NW = NC * NS
G = 512
KG = G // 128


def _sine_pos(seq_len, d, max_wavelength=10000.0):
    position = jnp.arange(seq_len, dtype=jnp.float32)
    min_freq = 1.0 / max_wavelength
    timescales = jnp.power(
        min_freq,
        (2.0 * (jnp.arange(d, dtype=jnp.float32) // 2)) / float(d),
    )
    angles = position[:, None] * timescales[None, :]
    cos_mask = (jnp.arange(d) % 2).astype(jnp.float32)
    sin_mask = 1.0 - cos_mask
    return jnp.sin(angles) * sin_mask + jnp.cos(angles) * cos_mask


def kernel(input, table):
    B, S = input.shape
    V, d = table.shape
    N = B * S
    n_w = N // NW
    T = n_w // G
    assert n_w % G == 0

    pos = _sine_pos(S, d)
    reps = (G + S - 1) // S + 1
    pos_t = jnp.tile(pos, (reps, 1))
    step_mod = G % S

    idx2 = input.reshape(N // 128, 128)

    mesh = plsc.VectorSubcoreMesh(core_axis_name="c", subcore_axis_name="s")

    @functools.partial(
        pl.kernel,
        mesh=mesh,
        out_type=jax.ShapeDtypeStruct((N, d), jnp.float32),
        scratch_types=[
            pltpu.VMEM((KG, 128), jnp.int32),
            pltpu.VMEM((G, d), jnp.float32),
            pltpu.VMEM((reps * S, d), jnp.float32),
            pltpu.SemaphoreType.DMA,
        ],
    )
    def sc_embed(idx_hbm, tab_hbm, pos_hbm, out_hbm, idx_v, rows_v, pos_v, sem):
        wid = lax.axis_index("s") * NC + lax.axis_index("c")
        base = wid * n_w
        j_base = wid * (n_w // 128)
        pltpu.sync_copy(pos_hbm, pos_v)

        def step(t, s0):
            row0 = base + t * G
            pltpu.sync_copy(idx_hbm.at[pl.ds(j_base + t * KG, KG)], idx_v)
            cps = [
                pltpu.async_copy(
                    tab_hbm.at[idx_v.at[k]],
                    rows_v.at[pl.ds(k * 128, 128)],
                    sem,
                )
                for k in range(KG)
            ]
            for c in cps:
                c.wait()

            def rbody(i, _):
                r = i * 4
                for u in range(4):
                    for h in (0, 16):
                        v = rows_v[r + u, pl.ds(h, 16)] + pos_v[s0 + r + u, pl.ds(h, 16)]
                        rows_v[r + u, pl.ds(h, 16)] = v
                return 0

            lax.fori_loop(0, G // 4, rbody, 0)
            pltpu.sync_copy(rows_v, out_hbm.at[pl.ds(row0, G)])
            s1 = s0 + step_mod
            return jnp.where(s1 >= S, s1 - S, s1)

        lax.fori_loop(0, T, step, jnp.int32(0))

    out = sc_embed(idx2, table, pos_t)
    return out.reshape(B, S, d)


# speedup vs baseline: 1.1252x; 1.1252x over previous
import functools

import jax
import jax.numpy as jnp
from jax import lax
from jax.experimental import pallas as pl
from jax.experimental.pallas import tpu as pltpu
from jax.experimental.pallas import tpu_sc as plsc

D = 32
NC = 2
NS = 16
NW = NC * NS
G = 512
KG = G // 128


def _sine_pos(seq_len, d, max_wavelength=10000.0):
    position = jnp.arange(seq_len, dtype=jnp.float32)
    min_freq = 1.0 / max_wavelength
    timescales = jnp.power(
        min_freq,
        (2.0 * (jnp.arange(d, dtype=jnp.float32) // 2)) / float(d),
    )
    angles = position[:, None] * timescales[None, :]
    cos_mask = (jnp.arange(d) % 2).astype(jnp.float32)
    sin_mask = 1.0 - cos_mask
    return jnp.sin(angles) * sin_mask + jnp.cos(angles) * cos_mask


def kernel(input, table):
    B, S = input.shape
    V, d = table.shape
    N = B * S
    n_w = N // NW
    T = n_w // G
    assert n_w % G == 0

    pos = _sine_pos(S, d)
    reps = (G + S - 1) // S + 1
    pos_t = jnp.tile(pos, (reps, 1))
    step_mod = G % S

    idx2 = input.reshape(N // 128, 128)

    mesh = plsc.VectorSubcoreMesh(core_axis_name="c", subcore_axis_name="s")

    @functools.partial(
        pl.kernel,
        mesh=mesh,
        compiler_params=pltpu.CompilerParams(use_tc_tiling_on_sc=False),
        out_type=jax.ShapeDtypeStruct((N, d), jnp.float32),
        scratch_types=[
            pltpu.VMEM((KG, 128), jnp.int32),
            pltpu.VMEM((G, d), jnp.float32),
            pltpu.VMEM((reps * S, d), jnp.float32),
            pltpu.SemaphoreType.DMA,
        ],
    )
    def sc_embed(idx_hbm, tab_hbm, pos_hbm, out_hbm, idx_v, rows_v, pos_v, sem):
        wid = lax.axis_index("s") * NC + lax.axis_index("c")
        base = wid * n_w
        j_base = wid * (n_w // 128)
        pltpu.sync_copy(pos_hbm, pos_v)

        def step(t, s0):
            row0 = base + t * G
            pltpu.sync_copy(idx_hbm.at[pl.ds(j_base + t * KG, KG)], idx_v)
            cps = [
                pltpu.async_copy(
                    tab_hbm.at[idx_v.at[k]],
                    rows_v.at[pl.ds(k * 128, 128)],
                    sem,
                )
                for k in range(KG)
            ]
            for c in cps:
                c.wait()

            def rbody(i, _):
                r = i * 4
                for u in range(4):
                    for h in (0, 16):
                        v = rows_v[r + u, pl.ds(h, 16)] + pos_v[s0 + r + u, pl.ds(h, 16)]
                        rows_v[r + u, pl.ds(h, 16)] = v
                return 0

            lax.fori_loop(0, G // 4, rbody, 0)
            pltpu.sync_copy(rows_v, out_hbm.at[pl.ds(row0, G)])
            s1 = s0 + step_mod
            return jnp.where(s1 >= S, s1 - S, s1)

        lax.fori_loop(0, T, step, jnp.int32(0))

    out = sc_embed(idx2, table, pos_t)
    return out.reshape(B, S, d)


# speedup vs baseline: 1.4365x; 1.2767x over previous
import functools

import jax
import jax.numpy as jnp
from jax import lax
from jax.experimental import pallas as pl
from jax.experimental.pallas import tpu as pltpu
from jax.experimental.pallas import tpu_sc as plsc

D = 32
NC = 2
NS = 16
NW = NC * NS
G = 512
KG = G // 128
NB = 5


def _sine_pos(seq_len, d, max_wavelength=10000.0):
    position = jnp.arange(seq_len, dtype=jnp.float32)
    min_freq = 1.0 / max_wavelength
    timescales = jnp.power(
        min_freq,
        (2.0 * (jnp.arange(d, dtype=jnp.float32) // 2)) / float(d),
    )
    angles = position[:, None] * timescales[None, :]
    cos_mask = (jnp.arange(d) % 2).astype(jnp.float32)
    sin_mask = 1.0 - cos_mask
    return jnp.sin(angles) * sin_mask + jnp.cos(angles) * cos_mask


def kernel(input, table):
    B, S = input.shape
    V, d = table.shape
    N = B * S
    n_w = N // NW
    T = n_w // G
    assert n_w % G == 0 and T % NB == 0 and T >= 2 * NB

    pos = _sine_pos(S, d)
    reps = (G + S - 1) // S + 1
    pos_t = jnp.tile(pos, (reps, 1))
    step_mod = G % S

    idx2 = input.reshape(N // 128, 128)

    mesh = plsc.VectorSubcoreMesh(core_axis_name="c", subcore_axis_name="s")

    @functools.partial(
        pl.kernel,
        mesh=mesh,
        compiler_params=pltpu.CompilerParams(use_tc_tiling_on_sc=False),
        out_type=jax.ShapeDtypeStruct((N, d), jnp.float32),
        scratch_types=[
            pltpu.VMEM((NB, KG, 128), jnp.int32),
            [pltpu.VMEM((G, d), jnp.float32) for _ in range(NB)],
            pltpu.VMEM_SHARED((reps * S, d), jnp.float32),
            [pltpu.SemaphoreType.DMA for _ in range(NB)],
            [pltpu.SemaphoreType.DMA for _ in range(NB)],
        ],
    )
    def sc_embed(idx_hbm, tab_hbm, pos_hbm, out_hbm, idx_v, rows, pos_v,
                 gsem, osem):
        wid = lax.axis_index("s") * NC + lax.axis_index("c")
        base = wid * n_w
        j_base = wid * (n_w // 128)

        @pl.when(lax.axis_index("s") == 0)
        def _stage_pos():
            pltpu.sync_copy(pos_hbm, pos_v)

        plsc.subcore_barrier()

        def fire_gathers(b, t_j):
            pltpu.sync_copy(idx_hbm.at[pl.ds(j_base + t_j * KG, KG)],
                            idx_v.at[b])
            for k in range(KG):
                pltpu.async_copy(
                    tab_hbm.at[idx_v.at[b].at[k]],
                    rows[b].at[pl.ds(k * 128, 128)],
                    gsem[b],
                    add=True,
                )

        def wait_gathers(b):
            pltpu.make_async_copy(
                tab_hbm.at[pl.ds(0, G)], rows[b], gsem[b]).wait()

        def fire_scatter(b, t_j):
            pltpu.async_copy(rows[b],
                             out_hbm.at[pl.ds(base + t_j * G, G)],
                             osem[b])

        def wait_scatter(b):
            pltpu.make_async_copy(rows[b], out_hbm.at[pl.ds(0, G)],
                                  osem[b]).wait()

        def prefill(b, sb):
            pltpu.sync_copy(pos_v.at[pl.ds(sb, G)], rows[b])

        for b in range(NB):
            prefill(b, (b * step_mod) % S)
            fire_gathers(b, jnp.int32(b))

        adv = [(b * step_mod) % S for b in range(NB)]
        adv_it = (NB * step_mod) % S

        def pair(p, s_prep):
            t0 = p * NB
            for b in range(NB):
                t = t0 + b
                wait_gathers(b)
                fire_scatter(b, t)
                sb = s_prep + adv[b]
                sb = jnp.where(sb >= S, sb - S, sb)
                wait_scatter(b)
                prefill(b, sb)
                fire_gathers(b, t + NB)
            s1 = s_prep + adv_it
            return jnp.where(s1 >= S, s1 - S, s1)

        lax.fori_loop(0, (T - NB) // NB, pair, jnp.int32(step_mod * NB % S))

        for b in range(NB):
            t = T - NB + b
            wait_gathers(b)
            fire_scatter(b, jnp.int32(t))
        for b in range(NB):
            wait_scatter(b)

    out = sc_embed(idx2, table, pos_t)
    return out.reshape(B, S, d)
